# baseline (speedup 1.0000x reference)
import jax
import jax.numpy as jnp
from jax import lax
from jax.experimental import pallas as pl
from jax.experimental.pallas import tpu as pltpu, tpu_sc as plsc

NUM_CORES = 2
NUM_SUBCORES = 16
NUM_WORKERS = NUM_CORES * NUM_SUBCORES
CHUNK = 128


def _embed_body(ids_hbm, table_hbm, out_hbm, idx_v, rows_v, sem):
    wid = lax.axis_index("s") * NUM_CORES + lax.axis_index("c")
    n = ids_hbm.shape[0]
    per_worker = n // NUM_WORKERS
    base = wid * per_worker
    nchunks = per_worker // CHUNK

    def step(ci, carry):
        off = base + ci * CHUNK
        pltpu.sync_copy(ids_hbm.at[pl.ds(off, CHUNK)], idx_v)
        pltpu.async_copy(table_hbm.at[idx_v], rows_v, sem).wait()
        pltpu.sync_copy(rows_v, out_hbm.at[pl.ds(off, CHUNK)])
        return carry

    lax.fori_loop(0, nchunks, step, 0)


def kernel(token_ids, table):
    batch, seq = token_ids.shape
    _, dim = table.shape
    ids = token_ids.reshape(-1).astype(jnp.int32)
    n = ids.shape[0]

    mesh = plsc.VectorSubcoreMesh(
        core_axis_name="c", subcore_axis_name="s",
        num_cores=NUM_CORES, num_subcores=NUM_SUBCORES,
    )
    out = pl.kernel(
        _embed_body,
        out_type=jax.ShapeDtypeStruct((n, dim), jnp.float32),
        mesh=mesh,
        scratch_types=[
            pltpu.VMEM((CHUNK,), jnp.int32),
            pltpu.VMEM((CHUNK, dim), jnp.float32),
            pltpu.SemaphoreType.DMA,
        ],
    )(ids, table)
    return out.reshape(batch, seq, dim)


# speedup vs baseline: 1.0093x; 1.0093x over previous
import jax
import jax.numpy as jnp
from jax import lax
from jax.experimental import pallas as pl
from jax.experimental.pallas import tpu as pltpu, tpu_sc as plsc

NUM_CORES = 2
NUM_SUBCORES = 16
NUM_WORKERS = NUM_CORES * NUM_SUBCORES
CHUNK = 80
NBUF = 2


def _embed_body(ids_hbm, table_hbm, out_hbm,
                idx_all, rows0, rows1, gsem0, gsem1, ssem0, ssem1):
    rows = (rows0, rows1)
    gsem = (gsem0, gsem1)
    ssem = (ssem0, ssem1)
    wid = lax.axis_index("s") * NUM_CORES + lax.axis_index("c")
    n = ids_hbm.shape[0]
    per_worker = n // NUM_WORKERS
    base = wid * per_worker
    nch = per_worker // CHUNK

    pltpu.sync_copy(ids_hbm.at[pl.ds(base, per_worker)], idx_all)

    def start_gather(ci, b):
        pltpu.async_copy(
            table_hbm.at[idx_all.at[pl.ds(ci * CHUNK, CHUNK)]], rows[b], gsem[b])

    def wait_gather(b):
        pltpu.make_async_copy(out_hbm.at[pl.ds(0, CHUNK)], rows[b], gsem[b]).wait()

    def wait_scatter(b):
        pltpu.make_async_copy(rows[b], out_hbm.at[pl.ds(0, CHUNK)], ssem[b]).wait()

    for b in range(NBUF):
        start_gather(b, b)

    @pl.loop(0, nch, step=NBUF)
    def _pipeline(i):
        for b in range(NBUF):
            ci = i + b
            wait_gather(b)
            pltpu.async_copy(
                rows[b], out_hbm.at[pl.ds(base + ci * CHUNK, CHUNK)], ssem[b])

            @pl.when(ci + NBUF < nch)
            def _refill():
                wait_scatter(b)
                start_gather(ci + NBUF, b)

    for b in range(NBUF):
        wait_scatter(b)


def kernel(token_ids, table):
    batch, seq = token_ids.shape
    _, dim = table.shape
    ids = token_ids.reshape(-1).astype(jnp.int32)
    n = ids.shape[0]
    per_worker = n // NUM_WORKERS

    mesh = plsc.VectorSubcoreMesh(
        core_axis_name="c", subcore_axis_name="s",
        num_cores=NUM_CORES, num_subcores=NUM_SUBCORES,
    )
    out = pl.kernel(
        _embed_body,
        out_type=jax.ShapeDtypeStruct((n, dim), jnp.float32),
        mesh=mesh,
        scratch_types=[
            pltpu.VMEM((per_worker,), jnp.int32),
            pltpu.VMEM((CHUNK, dim), jnp.float32),
            pltpu.VMEM((CHUNK, dim), jnp.float32),
            pltpu.SemaphoreType.DMA,
            pltpu.SemaphoreType.DMA,
            pltpu.SemaphoreType.DMA,
            pltpu.SemaphoreType.DMA,
        ],
    )(ids, table)
    return out.reshape(batch, seq, dim)
